# baseline (device time: 30661 ns/iter reference)
import jax
import jax.numpy as jnp
from jax import lax
from jax.experimental import pallas as pl
from jax.experimental.pallas import tpu as pltpu

T = 512
D = 512
F = 1024
E_LOCAL = 2


def kernel(x, assign, W1, W2):
    assign2d = assign.reshape(T, 1)

    def body(x_ref, a_ref, w1_ref, w2_ref, out_ref,
             xs, xr, as_s, as_r, ps, pr, send_sems, recv_sems):
        my_x = lax.axis_index("x")
        my_y = lax.axis_index("y")
        my_z = lax.axis_index("z")
        partner = (1 - my_x, my_y, my_z)

        barrier = pltpu.get_barrier_semaphore()
        pl.semaphore_signal(barrier, inc=1, device_id=partner,
                            device_id_type=pl.DeviceIdType.MESH)
        pl.semaphore_wait(barrier, 1)

        xs[...] = x_ref[...].astype(jnp.bfloat16)
        as_s[...] = a_ref[...]
        rdma_x = pltpu.make_async_remote_copy(
            src_ref=xs, dst_ref=xr,
            send_sem=send_sems.at[0], recv_sem=recv_sems.at[0],
            device_id=partner, device_id_type=pl.DeviceIdType.MESH)
        rdma_a = pltpu.make_async_remote_copy(
            src_ref=as_s, dst_ref=as_r,
            send_sem=send_sems.at[1], recv_sem=recv_sems.at[1],
            device_id=partner, device_id_type=pl.DeviceIdType.MESH)
        rdma_x.start()
        rdma_a.start()

        def ffn(xb, le):
            w1 = w1_ref[le].astype(jnp.bfloat16)
            w2 = w2_ref[le].astype(jnp.bfloat16)
            h = jnp.maximum(
                jnp.dot(xb, w1, preferred_element_type=jnp.float32), 0.0)
            return jnp.dot(h.astype(jnp.bfloat16), w2,
                           preferred_element_type=jnp.float32)

        def moe(xb, a2d):
            acc = jnp.zeros((T, D), jnp.float32)
            for le in range(E_LOCAL):
                e = my_x * E_LOCAL + le
                mask = (a2d == e).astype(jnp.float32)
                acc = acc + mask * ffn(xb, le)
            return acc

        out_ref[...] = moe(x_ref[...].astype(jnp.bfloat16), a_ref[...])

        rdma_x.wait()
        rdma_a.wait()

        ps[...] = moe(xr[...], as_r[...]).astype(jnp.bfloat16)
        rdma_p = pltpu.make_async_remote_copy(
            src_ref=ps, dst_ref=pr,
            send_sem=send_sems.at[2], recv_sem=recv_sems.at[2],
            device_id=partner, device_id_type=pl.DeviceIdType.MESH)
        rdma_p.start()
        rdma_p.wait()

        out_ref[...] = out_ref[...] + pr[...].astype(jnp.float32)

    return pl.pallas_call(
        body,
        out_shape=jax.ShapeDtypeStruct((T, D), jnp.float32),
        in_specs=[pl.BlockSpec(memory_space=pltpu.VMEM)] * 4,
        out_specs=pl.BlockSpec(memory_space=pltpu.VMEM),
        scratch_shapes=[
            pltpu.VMEM((T, D), jnp.bfloat16),
            pltpu.VMEM((T, D), jnp.bfloat16),
            pltpu.VMEM((T, 1), jnp.int32),
            pltpu.VMEM((T, 1), jnp.int32),
            pltpu.VMEM((T, D), jnp.bfloat16),
            pltpu.VMEM((T, D), jnp.bfloat16),
            pltpu.SemaphoreType.DMA((3,)),
            pltpu.SemaphoreType.DMA((3,)),
        ],
        compiler_params=pltpu.CompilerParams(collective_id=0),
    )(x, assign2d, W1, W2)


# device time: 29549 ns/iter; 1.0376x vs baseline; 1.0376x over previous
import jax
import jax.numpy as jnp
from jax import lax
from jax.experimental import pallas as pl
from jax.experimental.pallas import tpu as pltpu

T = 512
D = 512
F = 1024
E_LOCAL = 2
N_CHUNKS = 2
CHUNK = T // N_CHUNKS


def kernel(x, assign, W1, W2):
    assign2d = assign.reshape(T, 1)

    def body(x_ref, a_ref, w1_ref, w2_ref, out_ref,
             xs, xr, as_s, as_r, ps, pr, w1b, w2b, send_sems, recv_sems):
        my_x = lax.axis_index("x")
        my_y = lax.axis_index("y")
        my_z = lax.axis_index("z")
        partner = (1 - my_x, my_y, my_z)

        barrier = pltpu.get_barrier_semaphore()
        pl.semaphore_signal(barrier, inc=1, device_id=partner,
                            device_id_type=pl.DeviceIdType.MESH)
        pl.semaphore_wait(barrier, 1)

        xs[...] = x_ref[...].astype(jnp.bfloat16)
        as_s[...] = a_ref[...]
        rdma_x = pltpu.make_async_remote_copy(
            src_ref=xs, dst_ref=xr,
            send_sem=send_sems.at[0], recv_sem=recv_sems.at[0],
            device_id=partner, device_id_type=pl.DeviceIdType.MESH)
        rdma_a = pltpu.make_async_remote_copy(
            src_ref=as_s, dst_ref=as_r,
            send_sem=send_sems.at[1], recv_sem=recv_sems.at[1],
            device_id=partner, device_id_type=pl.DeviceIdType.MESH)
        rdma_x.start()
        rdma_a.start()

        w1b[...] = w1_ref[...].astype(jnp.bfloat16)
        w2b[...] = w2_ref[...].astype(jnp.bfloat16)

        def ffn(xb, le):
            h = jnp.maximum(
                jnp.dot(xb, w1b[le], preferred_element_type=jnp.float32), 0.0)
            return jnp.dot(h.astype(jnp.bfloat16), w2b[le],
                           preferred_element_type=jnp.float32)

        def masked(xb, a2d, le):
            e = my_x * E_LOCAL + le
            mask = (a2d == e).astype(jnp.float32)
            return mask * ffn(xb, le)

        xl = x_ref[...].astype(jnp.bfloat16)

        out_ref[...] = masked(xl, a_ref[...], 0)

        rdma_x.wait()
        rdma_a.wait()

        rdma_p = []
        for c in range(N_CHUNKS):
            rows = pl.ds(c * CHUNK, CHUNK)
            xc = xr[rows, :]
            ac = as_r[rows, :]
            acc = masked(xc, ac, 0) + masked(xc, ac, 1)
            ps[rows, :] = acc.astype(jnp.bfloat16)
            r = pltpu.make_async_remote_copy(
                src_ref=ps.at[rows, :], dst_ref=pr.at[rows, :],
                send_sem=send_sems.at[2 + c], recv_sem=recv_sems.at[2 + c],
                device_id=partner, device_id_type=pl.DeviceIdType.MESH)
            r.start()
            rdma_p.append(r)

        out_ref[...] = out_ref[...] + masked(xl, a_ref[...], 1)

        for r in rdma_p:
            r.wait()
        out_ref[...] = out_ref[...] + pr[...].astype(jnp.float32)

    return pl.pallas_call(
        body,
        out_shape=jax.ShapeDtypeStruct((T, D), jnp.float32),
        in_specs=[pl.BlockSpec(memory_space=pltpu.VMEM)] * 4,
        out_specs=pl.BlockSpec(memory_space=pltpu.VMEM),
        scratch_shapes=[
            pltpu.VMEM((T, D), jnp.bfloat16),
            pltpu.VMEM((T, D), jnp.bfloat16),
            pltpu.VMEM((T, 1), jnp.int32),
            pltpu.VMEM((T, 1), jnp.int32),
            pltpu.VMEM((T, D), jnp.bfloat16),
            pltpu.VMEM((T, D), jnp.bfloat16),
            pltpu.VMEM((E_LOCAL, D, F), jnp.bfloat16),
            pltpu.VMEM((E_LOCAL, F, D), jnp.bfloat16),
            pltpu.SemaphoreType.DMA((2 + N_CHUNKS,)),
            pltpu.SemaphoreType.DMA((2 + N_CHUNKS,)),
        ],
        compiler_params=pltpu.CompilerParams(collective_id=0),
    )(x, assign2d, W1, W2)
